# baseline (speedup 1.0000x reference)
import functools

import jax
import jax.numpy as jnp
from jax import lax
from jax.experimental import pallas as pl
from jax.experimental.pallas import tpu as pltpu
from jax.experimental.pallas import tpu_sc as plsc

NC = 2
NS = 16
NW = NC * NS
L = 16
CHUNK = 128


def _make_kernel(B, D):
    b_per_w = B // NW
    n_chunk = b_per_w // CHUNK
    mesh = plsc.VectorSubcoreMesh(core_axis_name="c", subcore_axis_name="s")

    @functools.partial(
        pl.kernel,
        mesh=mesh,
        compiler_params=pltpu.CompilerParams(
            needs_layout_passes=False, use_tc_tiling_on_sc=False
        ),
        out_type=jax.ShapeDtypeStruct((B,), jnp.float32),
        scratch_types=[
            pltpu.VMEM((n_chunk, CHUNK), jnp.int32),
            pltpu.VMEM((n_chunk, CHUNK), jnp.int32),
            pltpu.VMEM((n_chunk, CHUNK), jnp.int32),
            pltpu.VMEM((b_per_w, D), jnp.float32),
            pltpu.VMEM((b_per_w, D), jnp.float32),
            pltpu.VMEM((b_per_w, D), jnp.float32),
            pltpu.VMEM((b_per_w,), jnp.float32),
            pltpu.SemaphoreType.DMA,
        ],
    )
    def k(node_hbm, rel_hbm, sidx_hbm, ridx_hbm, oidx_hbm, out_hbm,
          sidx_v, ridx_v, oidx_v, s_v, r_v, o_v, out_v, sem):
        wid = lax.axis_index("s") * NC + lax.axis_index("c")
        crow = wid * n_chunk
        pltpu.sync_copy(sidx_hbm.at[pl.ds(crow, n_chunk)], sidx_v)
        pltpu.sync_copy(ridx_hbm.at[pl.ds(crow, n_chunk)], ridx_v)
        pltpu.sync_copy(oidx_hbm.at[pl.ds(crow, n_chunk)], oidx_v)
        cps = []
        for j in range(n_chunk):
            dst = pl.ds(j * CHUNK, CHUNK)
            cps.append(pltpu.async_copy(node_hbm.at[sidx_v.at[j]], s_v.at[dst], sem))
            cps.append(pltpu.async_copy(rel_hbm.at[ridx_v.at[j]], r_v.at[dst], sem))
            cps.append(pltpu.async_copy(node_hbm.at[oidx_v.at[j]], o_v.at[dst], sem))
        for cp in cps:
            cp.wait()

        def body(g, carry):
            i0 = g * L
            rows = i0 + lax.iota(jnp.int32, L)
            accs = [jnp.zeros((L,), jnp.float32) for _ in range(4)]
            for d in range(D):
                cols = jnp.full((L,), d, jnp.int32)
                sv = plsc.load_gather(s_v, [rows, cols])
                rv = plsc.load_gather(r_v, [rows, cols])
                ov = plsc.load_gather(o_v, [rows, cols])
                accs[d % 4] = accs[d % 4] + sv * rv * ov
            out_v[pl.ds(i0, L)] = (accs[0] + accs[1]) + (accs[2] + accs[3])
            return carry

        lax.fori_loop(0, b_per_w // L, body, 0)
        pltpu.sync_copy(out_v, out_hbm.at[pl.ds(wid * b_per_w, b_per_w)])

    return k


def kernel(node_embeddings, rel_embeddings, triplets):
    B = triplets.shape[0]
    D = node_embeddings.shape[1]
    idx = triplets.astype(jnp.int32)
    sidx = idx[:, 0].reshape(B // CHUNK, CHUNK)
    ridx = idx[:, 1].reshape(B // CHUNK, CHUNK)
    oidx = idx[:, 2].reshape(B // CHUNK, CHUNK)
    return _make_kernel(B, D)(node_embeddings, rel_embeddings, sidx, ridx, oidx)


# speedup vs baseline: 2.2099x; 2.2099x over previous
import functools

import jax
import jax.numpy as jnp
from jax import lax
from jax.experimental import pallas as pl
from jax.experimental.pallas import tpu as pltpu
from jax.experimental.pallas import tpu_sc as plsc

NC = 2
NS = 16
NW = NC * NS
L = 16
RB = 8


def _make_kernel(B, D):
    b_per_w = B // NW
    n_grp = b_per_w // L
    assert n_grp % 2 == 0
    mesh = plsc.VectorSubcoreMesh(core_axis_name="c", subcore_axis_name="s")
    blk = pltpu.VMEM((L, RB, D), jnp.float32)

    @functools.partial(
        pl.kernel,
        mesh=mesh,
        compiler_params=pltpu.CompilerParams(needs_layout_passes=False),
        out_type=jax.ShapeDtypeStruct((B,), jnp.float32),
        scratch_types=[
            pltpu.VMEM((b_per_w,), jnp.int32),
            pltpu.VMEM((b_per_w,), jnp.int32),
            pltpu.VMEM((b_per_w,), jnp.int32),
            blk, blk, blk,
            blk, blk, blk,
            pltpu.VMEM((b_per_w,), jnp.float32),
            pltpu.SemaphoreType.DMA,
            pltpu.SemaphoreType.DMA,
        ],
    )
    def k(node_hbm, rel_hbm, sidx_hbm, ridx_hbm, oidx_hbm, out_hbm,
          sidx_v, ridx_v, oidx_v, sA, rA, oA, sB, rB_, oB, out_v,
          semA, semB):
        wid = lax.axis_index("s") * NC + lax.axis_index("c")
        base = wid * b_per_w
        pltpu.sync_copy(sidx_hbm.at[pl.ds(base, b_per_w)], sidx_v)
        pltpu.sync_copy(ridx_hbm.at[pl.ds(base, b_per_w)], ridx_v)
        pltpu.sync_copy(oidx_hbm.at[pl.ds(base, b_per_w)], oidx_v)

        lanes = lax.iota(jnp.int32, L)

        def fire(c, sbuf, rbuf, obuf, sem):
            ts = lax.shift_right_logical(sidx_v[pl.ds(c * L, L)], 3)
            tr = lax.shift_right_logical(ridx_v[pl.ds(c * L, L)], 3)
            to = lax.shift_right_logical(oidx_v[pl.ds(c * L, L)], 3)
            for i in range(L):
                pltpu.async_copy(node_hbm.at[ts[i]], sbuf.at[i], sem)
                pltpu.async_copy(rel_hbm.at[tr[i]], rbuf.at[i], sem)
                pltpu.async_copy(node_hbm.at[to[i]], obuf.at[i], sem)

        def drain(sbuf, rbuf, obuf, sem):
            pltpu.make_async_copy(node_hbm.at[pl.ds(0, L)], sbuf, sem).wait()
            pltpu.make_async_copy(rel_hbm.at[pl.ds(0, L)], rbuf, sem).wait()
            pltpu.make_async_copy(node_hbm.at[pl.ds(0, L)], obuf, sem).wait()

        def compute(c, sbuf, rbuf, obuf):
            sub_s = lax.bitwise_and(sidx_v[pl.ds(c * L, L)], RB - 1)
            sub_r = lax.bitwise_and(ridx_v[pl.ds(c * L, L)], RB - 1)
            sub_o = lax.bitwise_and(oidx_v[pl.ds(c * L, L)], RB - 1)
            accs = [jnp.zeros((L,), jnp.float32) for _ in range(4)]
            for d in range(D):
                cols = jnp.full((L,), d, jnp.int32)
                sv = plsc.load_gather(sbuf, [lanes, sub_s, cols])
                rv = plsc.load_gather(rbuf, [lanes, sub_r, cols])
                ov = plsc.load_gather(obuf, [lanes, sub_o, cols])
                accs[d % 4] = accs[d % 4] + sv * rv * ov
            out_v[pl.ds(c * L, L)] = (accs[0] + accs[1]) + (accs[2] + accs[3])

        fire(0, sA, rA, oA, semA)

        def outer(h, carry):
            g = h * 2
            fire(g + 1, sB, rB_, oB, semB)
            drain(sA, rA, oA, semA)
            compute(g, sA, rA, oA)

            @pl.when(g + 2 < n_grp)
            def _():
                fire(g + 2, sA, rA, oA, semA)

            drain(sB, rB_, oB, semB)
            compute(g + 1, sB, rB_, oB)
            return carry

        lax.fori_loop(0, n_grp // 2, outer, 0)
        pltpu.sync_copy(out_v, out_hbm.at[pl.ds(base, b_per_w)])

    return k


def kernel(node_embeddings, rel_embeddings, triplets):
    B = triplets.shape[0]
    V, D = node_embeddings.shape
    R = rel_embeddings.shape[0]
    idx = triplets.astype(jnp.int32)
    node3 = node_embeddings.reshape(V // RB, RB, D)
    rel3 = rel_embeddings.reshape(R // RB, RB, D)
    return _make_kernel(B, D)(node3, rel3, idx[:, 0], idx[:, 1], idx[:, 2])
